# Initial kernel scaffold; baseline (speedup 1.0000x reference)
import functools

import jax
import jax.numpy as jnp
from jax import lax
from jax.experimental import pallas as pl
from jax.experimental.pallas import tpu as pltpu
from jax.experimental.pallas import tpu_sc as plsc


def _gather_sc(idx_flat, table, n, d, num_workers, chunk):
    per_w = n // num_workers
    n_chunks = per_w // chunk
    mesh = plsc.VectorSubcoreMesh(core_axis_name="c", subcore_axis_name="s")

    @functools.partial(
        pl.kernel,
        mesh=mesh,
        out_type=jax.ShapeDtypeStruct((n, d), jnp.float32),
        scratch_types=[
            pltpu.VMEM((per_w,), jnp.int32),
            pltpu.VMEM((2, chunk, d), jnp.float32),
            pltpu.SemaphoreType.DMA,
            pltpu.SemaphoreType.DMA,
        ],
    )
    def k(idx_hbm, table_hbm, out_hbm, idx_v, rows_v, gsem, ssem):
        wid = lax.axis_index("s") * 2 + lax.axis_index("c")
        base = wid * per_w
        pltpu.sync_copy(idx_hbm.at[pl.ds(base, per_w)], idx_v)

        def body(c, _):
            off = c * chunk
            pltpu.async_copy(
                table_hbm.at[idx_v.at[pl.ds(off, chunk)]],
                rows_v.at[0],
                gsem,
            ).wait()
            pltpu.async_copy(
                rows_v.at[0],
                out_hbm.at[pl.ds(base + off, chunk)],
                ssem,
            ).wait()
            return 0

        lax.fori_loop(0, n_chunks, body, 0, unroll=False)

    return k(idx_flat, table)


def kernel(idx, token_embedding_table):
    b, t = idx.shape
    v, d = token_embedding_table.shape
    n = b * t
    out = _gather_sc(
        idx.reshape(n), token_embedding_table, n, d, num_workers=32, chunk=64
    )
    return out.reshape(b, t, d)


# speedup vs baseline: 1.0144x; 1.0144x over previous
import functools

import jax
import jax.numpy as jnp
from jax import lax
from jax.experimental import pallas as pl
from jax.experimental.pallas import tpu as pltpu
from jax.experimental.pallas import tpu_sc as plsc


def _gather_sc(idx_flat, table, n, d, num_workers, chunk):
    per_w = n // num_workers
    n_chunks = per_w // chunk
    mesh = plsc.VectorSubcoreMesh(core_axis_name="c", subcore_axis_name="s")

    @functools.partial(
        pl.kernel,
        mesh=mesh,
        out_type=jax.ShapeDtypeStruct((n, d), jnp.float32),
        compiler_params=pltpu.CompilerParams(use_tc_tiling_on_sc=False),
        scratch_types=[
            pltpu.VMEM((per_w,), jnp.int32),
            pltpu.VMEM((2, chunk, d), jnp.float32),
            pltpu.SemaphoreType.DMA,
            pltpu.SemaphoreType.DMA,
        ],
    )
    def k(idx_hbm, table_hbm, out_hbm, idx_v, rows_v, gsem, ssem):
        wid = lax.axis_index("s") * 2 + lax.axis_index("c")
        base = wid * per_w
        pltpu.sync_copy(idx_hbm.at[pl.ds(base, per_w)], idx_v)

        def body(c, _):
            off = c * chunk
            pltpu.async_copy(
                table_hbm.at[idx_v.at[pl.ds(off, chunk)]],
                rows_v.at[0],
                gsem,
            ).wait()
            pltpu.async_copy(
                rows_v.at[0],
                out_hbm.at[pl.ds(base + off, chunk)],
                ssem,
            ).wait()
            return 0

        lax.fori_loop(0, n_chunks, body, 0, unroll=False)

    return k(idx_flat, table)


def kernel(idx, token_embedding_table):
    b, t = idx.shape
    v, d = token_embedding_table.shape
    n = b * t
    out = _gather_sc(
        idx.reshape(n), token_embedding_table, n, d, num_workers=32, chunk=64
    )
    return out.reshape(b, t, d)


# speedup vs baseline: 1.0301x; 1.0154x over previous
import functools

import jax
import jax.numpy as jnp
from jax import lax
from jax.experimental import pallas as pl
from jax.experimental.pallas import tpu as pltpu
from jax.experimental.pallas import tpu_sc as plsc


def _gather_sc(idx_flat, table, n, d, num_workers, chunk, nbuf):
    per_w = n // num_workers
    n_chunks = per_w // chunk
    mesh = plsc.VectorSubcoreMesh(core_axis_name="c", subcore_axis_name="s")

    @functools.partial(
        pl.kernel,
        mesh=mesh,
        out_type=jax.ShapeDtypeStruct((n, d), jnp.float32),
        compiler_params=pltpu.CompilerParams(use_tc_tiling_on_sc=False),
        scratch_types=[
            pltpu.VMEM((per_w,), jnp.int32),
            pltpu.VMEM((nbuf, chunk, d), jnp.float32),
            pltpu.SemaphoreType.DMA((nbuf,)),
            pltpu.SemaphoreType.DMA((nbuf,)),
        ],
    )
    def k(idx_hbm, table_hbm, out_hbm, idx_v, rows_v, gsem, ssem):
        wid = lax.axis_index("s") * 2 + lax.axis_index("c")
        base = wid * per_w
        pltpu.sync_copy(idx_hbm.at[pl.ds(base, per_w)], idx_v)

        def start_gather(c):
            p = c % nbuf
            return pltpu.async_copy(
                table_hbm.at[idx_v.at[pl.ds(c * chunk, chunk)]],
                rows_v.at[p],
                gsem.at[p],
            )

        def start_scatter(c):
            p = c % nbuf
            return pltpu.async_copy(
                rows_v.at[p],
                out_hbm.at[pl.ds(base + c * chunk, chunk)],
                ssem.at[p],
            )

        gathers = [None] * n_chunks
        scatters = [None] * n_chunks
        gathers[0] = start_gather(0)
        for c in range(n_chunks):
            if c + 1 < n_chunks:
                if c + 1 >= nbuf:
                    scatters[c + 1 - nbuf].wait()
                gathers[c + 1] = start_gather(c + 1)
            gathers[c].wait()
            scatters[c] = start_scatter(c)
        for c in range(max(0, n_chunks - nbuf), n_chunks):
            scatters[c].wait()

    return k(idx_flat, table)


def kernel(idx, token_embedding_table):
    b, t = idx.shape
    v, d = token_embedding_table.shape
    n = b * t
    out = _gather_sc(
        idx.reshape(n), token_embedding_table, n, d,
        num_workers=32, chunk=64, nbuf=2,
    )
    return out.reshape(b, t, d)
